# baseline (speedup 1.0000x reference)
import functools

import jax
import jax.numpy as jnp
from jax.experimental import pallas as pl
from jax.experimental.pallas import tpu as pltpu


def _stage1_body(x1_ref, x2t_ref, p1_ref, p2_ref, w1at_ref, w1bt_ref, b1_ref,
                 y1_ref, s_ref, ss_ref, p2w_scr):
    b = pl.program_id(0)
    n = pl.program_id(1)

    @pl.when(n == 0)
    def _():
        p2w_scr[...] = jnp.dot(p2_ref[0], w1bt_ref[...],
                               preferred_element_type=jnp.float32, precision=jax.lax.Precision.HIGHEST)

    x1 = x1_ref[0]
    x2t = x2t_ref[0]

    def _weights(xh):
        d0 = xh[:, 0:1] - x2t[0:1, :]
        d1 = xh[:, 1:2] - x2t[1:2, :]
        d2c = xh[:, 2:3] - x2t[2:3, :]
        d = d0 * d0 + d1 * d1 + d2c * d2c
        inf = jnp.float32(jnp.inf)
        m1 = jnp.min(d, axis=1, keepdims=True)
        m2 = jnp.min(jnp.where(d <= m1, inf, d), axis=1, keepdims=True)
        m3 = jnp.min(jnp.where(d <= m2, inf, d), axis=1, keepdims=True)
        r = jnp.where(d <= m3, 1.0 / (d + 1e-8), 0.0)
        return r / jnp.sum(r, axis=1, keepdims=True)

    TH = x1.shape[0] // 2
    y1p = jnp.dot(p1_ref[0], w1at_ref[...],
                  preferred_element_type=jnp.float32,
                  precision=jax.lax.Precision.HIGHEST)
    halves = []
    for h in range(2):
        wt = _weights(x1[h * TH:(h + 1) * TH])
        halves.append(jnp.dot(wt, p2w_scr[...],
                              preferred_element_type=jnp.float32,
                              precision=jax.lax.Precision.HIGHEST))
    y1 = y1p + jnp.concatenate(halves, axis=0) + b1_ref[...]
    y1_ref[0] = y1

    @pl.when((b == 0) & (n == 0))
    def _():
        s_ref[...] = jnp.zeros_like(s_ref)
        ss_ref[...] = jnp.zeros_like(ss_ref)

    s_ref[...] += jnp.sum(y1, axis=0, keepdims=True)
    ss_ref[...] += jnp.sum(y1 * y1, axis=0, keepdims=True)


def _bn_affine(s, ss, g, beta, minv):
    mean = s * minv
    var = ss * minv - mean * mean
    inv = jax.lax.rsqrt(var + 1e-5)
    scale = g * inv
    shift = beta - mean * scale
    return scale, shift


def _stage2_body(y1_ref, s_ref, ss_ref, g1_ref, bt1_ref, w2t_ref, b2_ref,
                 y2_ref, s2_ref, ss2_ref, *, minv):
    b = pl.program_id(0)
    n = pl.program_id(1)
    scale, shift = _bn_affine(s_ref[...], ss_ref[...], g1_ref[...],
                              bt1_ref[...], minv)
    h = jnp.maximum(y1_ref[0] * scale + shift, 0.0)
    y2 = jnp.dot(h, w2t_ref[...], preferred_element_type=jnp.float32, precision=jax.lax.Precision.HIGHEST)
    y2 = y2 + b2_ref[...]
    y2_ref[0] = y2

    @pl.when((b == 0) & (n == 0))
    def _():
        s2_ref[...] = jnp.zeros_like(s2_ref)
        ss2_ref[...] = jnp.zeros_like(ss2_ref)

    s2_ref[...] += jnp.sum(y2, axis=0, keepdims=True)
    ss2_ref[...] += jnp.sum(y2 * y2, axis=0, keepdims=True)


def _stage3_body(y2_ref, s2_ref, ss2_ref, g2_ref, bt2_ref, out_ref, *, minv):
    scale, shift = _bn_affine(s2_ref[...], ss2_ref[...], g2_ref[...],
                              bt2_ref[...], minv)
    out_ref[0] = jnp.maximum(y2_ref[0] * scale + shift, 0.0)


def kernel(xyz1, xyz2, points1, points2, W1, b1, g1, beta1, W2, b2, g2, beta2):
    B, N, _ = xyz1.shape
    S = xyz2.shape[1]
    C1 = points1.shape[2]
    C2 = points2.shape[2]
    F1 = W1.shape[0]
    F2 = W2.shape[0]
    TN = 256
    NT = N // TN
    minv = 1.0 / (B * N)

    x2t = jnp.transpose(xyz2, (0, 2, 1))
    w1at = jnp.transpose(W1[:, :C1])
    w1bt = jnp.transpose(W1[:, C1:])
    w2t = jnp.transpose(W2)
    b1r = b1.reshape(1, F1)
    g1r = g1.reshape(1, F1)
    bt1r = beta1.reshape(1, F1)
    b2r = b2.reshape(1, F2)
    g2r = g2.reshape(1, F2)
    bt2r = beta2.reshape(1, F2)

    f32 = jnp.float32
    y1, s1, ss1 = pl.pallas_call(
        _stage1_body,
        grid=(B, NT),
        in_specs=[
            pl.BlockSpec((1, TN, 3), lambda b, n: (b, n, 0)),
            pl.BlockSpec((1, 3, S), lambda b, n: (b, 0, 0)),
            pl.BlockSpec((1, TN, C1), lambda b, n: (b, n, 0)),
            pl.BlockSpec((1, S, C2), lambda b, n: (b, 0, 0)),
            pl.BlockSpec((C1, F1), lambda b, n: (0, 0)),
            pl.BlockSpec((C2, F1), lambda b, n: (0, 0)),
            pl.BlockSpec((1, F1), lambda b, n: (0, 0)),
        ],
        out_specs=[
            pl.BlockSpec((1, TN, F1), lambda b, n: (b, n, 0)),
            pl.BlockSpec((1, F1), lambda b, n: (0, 0)),
            pl.BlockSpec((1, F1), lambda b, n: (0, 0)),
        ],
        out_shape=[
            jax.ShapeDtypeStruct((B, N, F1), f32),
            jax.ShapeDtypeStruct((1, F1), f32),
            jax.ShapeDtypeStruct((1, F1), f32),
        ],
        scratch_shapes=[pltpu.VMEM((S, F1), f32)],
    )(xyz1, x2t, points1, points2, w1at, w1bt, b1r)

    y2, s2, ss2 = pl.pallas_call(
        functools.partial(_stage2_body, minv=minv),
        grid=(B, NT),
        in_specs=[
            pl.BlockSpec((1, TN, F1), lambda b, n: (b, n, 0)),
            pl.BlockSpec((1, F1), lambda b, n: (0, 0)),
            pl.BlockSpec((1, F1), lambda b, n: (0, 0)),
            pl.BlockSpec((1, F1), lambda b, n: (0, 0)),
            pl.BlockSpec((1, F1), lambda b, n: (0, 0)),
            pl.BlockSpec((F1, F2), lambda b, n: (0, 0)),
            pl.BlockSpec((1, F2), lambda b, n: (0, 0)),
        ],
        out_specs=[
            pl.BlockSpec((1, TN, F2), lambda b, n: (b, n, 0)),
            pl.BlockSpec((1, F2), lambda b, n: (0, 0)),
            pl.BlockSpec((1, F2), lambda b, n: (0, 0)),
        ],
        out_shape=[
            jax.ShapeDtypeStruct((B, N, F2), f32),
            jax.ShapeDtypeStruct((1, F2), f32),
            jax.ShapeDtypeStruct((1, F2), f32),
        ],
    )(y1, s1, ss1, g1r, bt1r, w2t, b2r)

    out = pl.pallas_call(
        functools.partial(_stage3_body, minv=minv),
        grid=(B, NT),
        in_specs=[
            pl.BlockSpec((1, TN, F2), lambda b, n: (b, n, 0)),
            pl.BlockSpec((1, F2), lambda b, n: (0, 0)),
            pl.BlockSpec((1, F2), lambda b, n: (0, 0)),
            pl.BlockSpec((1, F2), lambda b, n: (0, 0)),
            pl.BlockSpec((1, F2), lambda b, n: (0, 0)),
        ],
        out_specs=pl.BlockSpec((1, TN, F2), lambda b, n: (b, n, 0)),
        out_shape=jax.ShapeDtypeStruct((B, N, F2), f32),
    )(y2, s2, ss2, g2r, bt2r)

    return out


# speedup vs baseline: 1.2484x; 1.2484x over previous
import functools

import jax
import jax.numpy as jnp
from jax.experimental import pallas as pl
from jax.experimental.pallas import tpu as pltpu


def _stage1_body(x1_ref, x2t_ref, p1_ref, p2_ref, w1at_ref, w1bt_ref, b1_ref,
                 y1_ref, s_ref, ss_ref, p2w_scr):
    b = pl.program_id(0)
    n = pl.program_id(1)

    @pl.when(n == 0)
    def _():
        p2w_scr[...] = jnp.dot(p2_ref[0], w1bt_ref[...],
                               preferred_element_type=jnp.float32, precision=jax.lax.Precision.HIGHEST)

    x1 = x1_ref[0]
    x2t = x2t_ref[0]

    def _weights(xh):
        d0 = xh[:, 0:1] - x2t[0:1, :]
        d1 = xh[:, 1:2] - x2t[1:2, :]
        d2c = xh[:, 2:3] - x2t[2:3, :]
        d = d0 * d0 + d1 * d1 + d2c * d2c
        inf = jnp.float32(jnp.inf)
        m1 = jnp.min(d, axis=1, keepdims=True)
        m2 = jnp.min(jnp.where(d <= m1, inf, d), axis=1, keepdims=True)
        m3 = jnp.min(jnp.where(d <= m2, inf, d), axis=1, keepdims=True)
        r = jnp.where(d <= m3, 1.0 / (d + 1e-8), 0.0)
        return r / jnp.sum(r, axis=1, keepdims=True)

    TH = x1.shape[0] // 2
    y1p = jnp.dot(p1_ref[0], w1at_ref[...],
                  preferred_element_type=jnp.float32,
                  precision=jax.lax.Precision.HIGHEST)
    halves = []
    for h in range(2):
        wt = _weights(x1[h * TH:(h + 1) * TH])
        halves.append(jnp.dot(wt, p2w_scr[...],
                              preferred_element_type=jnp.float32,
                              precision=jax.lax.Precision.HIGHEST))
    y1 = y1p + jnp.concatenate(halves, axis=0) + b1_ref[...]
    y1_ref[0] = y1

    @pl.when((b == 0) & (n == 0))
    def _():
        s_ref[...] = jnp.zeros_like(s_ref)
        ss_ref[...] = jnp.zeros_like(ss_ref)

    s_ref[...] += jnp.sum(y1, axis=0, keepdims=True)
    ss_ref[...] += jnp.sum(y1 * y1, axis=0, keepdims=True)


def _bn_affine(s, ss, g, beta, minv):
    mean = s * minv
    var = ss * minv - mean * mean
    inv = jax.lax.rsqrt(var + 1e-5)
    scale = g * inv
    shift = beta - mean * scale
    return scale, shift


def _stage2_body(y1_ref, s_ref, ss_ref, g1_ref, bt1_ref, w2t_ref, b2_ref,
                 y2_ref, s2_ref, ss2_ref, *, minv):
    b = pl.program_id(0)
    n = pl.program_id(1)
    scale, shift = _bn_affine(s_ref[...], ss_ref[...], g1_ref[...],
                              bt1_ref[...], minv)
    h = jnp.maximum(y1_ref[0] * scale + shift, 0.0)
    y2 = jnp.dot(h, w2t_ref[...], preferred_element_type=jnp.float32, precision=jax.lax.Precision.HIGHEST)
    y2 = y2 + b2_ref[...]
    y2_ref[0] = y2

    @pl.when((b == 0) & (n == 0))
    def _():
        s2_ref[...] = jnp.zeros_like(s2_ref)
        ss2_ref[...] = jnp.zeros_like(ss2_ref)

    s2_ref[...] += jnp.sum(y2, axis=0, keepdims=True)
    ss2_ref[...] += jnp.sum(y2 * y2, axis=0, keepdims=True)


def _stage3_body(y2_ref, s2_ref, ss2_ref, g2_ref, bt2_ref, out_ref, *, minv):
    scale, shift = _bn_affine(s2_ref[...], ss2_ref[...], g2_ref[...],
                              bt2_ref[...], minv)
    out_ref[0] = jnp.maximum(y2_ref[0] * scale + shift, 0.0)


def kernel(xyz1, xyz2, points1, points2, W1, b1, g1, beta1, W2, b2, g2, beta2):
    B, N, _ = xyz1.shape
    S = xyz2.shape[1]
    C1 = points1.shape[2]
    C2 = points2.shape[2]
    F1 = W1.shape[0]
    F2 = W2.shape[0]
    TN = 512
    NT = N // TN
    minv = 1.0 / (B * N)

    x2t = jnp.transpose(xyz2, (0, 2, 1))
    w1at = jnp.transpose(W1[:, :C1])
    w1bt = jnp.transpose(W1[:, C1:])
    w2t = jnp.transpose(W2)
    b1r = b1.reshape(1, F1)
    g1r = g1.reshape(1, F1)
    bt1r = beta1.reshape(1, F1)
    b2r = b2.reshape(1, F2)
    g2r = g2.reshape(1, F2)
    bt2r = beta2.reshape(1, F2)

    f32 = jnp.float32
    y1, s1, ss1 = pl.pallas_call(
        _stage1_body,
        grid=(B, NT),
        in_specs=[
            pl.BlockSpec((1, TN, 3), lambda b, n: (b, n, 0)),
            pl.BlockSpec((1, 3, S), lambda b, n: (b, 0, 0)),
            pl.BlockSpec((1, TN, C1), lambda b, n: (b, n, 0)),
            pl.BlockSpec((1, S, C2), lambda b, n: (b, 0, 0)),
            pl.BlockSpec((C1, F1), lambda b, n: (0, 0)),
            pl.BlockSpec((C2, F1), lambda b, n: (0, 0)),
            pl.BlockSpec((1, F1), lambda b, n: (0, 0)),
        ],
        out_specs=[
            pl.BlockSpec((1, TN, F1), lambda b, n: (b, n, 0)),
            pl.BlockSpec((1, F1), lambda b, n: (0, 0)),
            pl.BlockSpec((1, F1), lambda b, n: (0, 0)),
        ],
        out_shape=[
            jax.ShapeDtypeStruct((B, N, F1), f32),
            jax.ShapeDtypeStruct((1, F1), f32),
            jax.ShapeDtypeStruct((1, F1), f32),
        ],
        scratch_shapes=[pltpu.VMEM((S, F1), f32)],
    )(xyz1, x2t, points1, points2, w1at, w1bt, b1r)

    y2, s2, ss2 = pl.pallas_call(
        functools.partial(_stage2_body, minv=minv),
        grid=(B, NT),
        in_specs=[
            pl.BlockSpec((1, TN, F1), lambda b, n: (b, n, 0)),
            pl.BlockSpec((1, F1), lambda b, n: (0, 0)),
            pl.BlockSpec((1, F1), lambda b, n: (0, 0)),
            pl.BlockSpec((1, F1), lambda b, n: (0, 0)),
            pl.BlockSpec((1, F1), lambda b, n: (0, 0)),
            pl.BlockSpec((F1, F2), lambda b, n: (0, 0)),
            pl.BlockSpec((1, F2), lambda b, n: (0, 0)),
        ],
        out_specs=[
            pl.BlockSpec((1, TN, F2), lambda b, n: (b, n, 0)),
            pl.BlockSpec((1, F2), lambda b, n: (0, 0)),
            pl.BlockSpec((1, F2), lambda b, n: (0, 0)),
        ],
        out_shape=[
            jax.ShapeDtypeStruct((B, N, F2), f32),
            jax.ShapeDtypeStruct((1, F2), f32),
            jax.ShapeDtypeStruct((1, F2), f32),
        ],
    )(y1, s1, ss1, g1r, bt1r, w2t, b2r)

    out = pl.pallas_call(
        functools.partial(_stage3_body, minv=minv),
        grid=(B, NT),
        in_specs=[
            pl.BlockSpec((1, TN, F2), lambda b, n: (b, n, 0)),
            pl.BlockSpec((1, F2), lambda b, n: (0, 0)),
            pl.BlockSpec((1, F2), lambda b, n: (0, 0)),
            pl.BlockSpec((1, F2), lambda b, n: (0, 0)),
            pl.BlockSpec((1, F2), lambda b, n: (0, 0)),
        ],
        out_specs=pl.BlockSpec((1, TN, F2), lambda b, n: (b, n, 0)),
        out_shape=jax.ShapeDtypeStruct((B, N, F2), f32),
    )(y2, s2, ss2, g2r, bt2r)

    return out


# speedup vs baseline: 1.3579x; 1.0878x over previous
import functools

import jax
import jax.numpy as jnp
from jax.experimental import pallas as pl
from jax.experimental.pallas import tpu as pltpu


def _stage1_body(x1_ref, x2t_ref, p1_ref, p2_ref, w1at_ref, w1bt_ref, b1_ref,
                 y1_ref, s_ref, ss_ref, p2w_scr):
    b = pl.program_id(0)
    n = pl.program_id(1)

    @pl.when(n == 0)
    def _():
        p2w_scr[...] = jnp.dot(p2_ref[0], w1bt_ref[...],
                               preferred_element_type=jnp.float32, precision=jax.lax.Precision.HIGHEST)

    x1 = x1_ref[0]
    x2t = x2t_ref[0]

    def _weights(xh):
        d0 = xh[:, 0:1] - x2t[0:1, :]
        d1 = xh[:, 1:2] - x2t[1:2, :]
        d2c = xh[:, 2:3] - x2t[2:3, :]
        d = d0 * d0 + d1 * d1 + d2c * d2c
        inf = jnp.float32(jnp.inf)
        m1 = jnp.min(d, axis=1, keepdims=True)
        m2 = jnp.min(jnp.where(d <= m1, inf, d), axis=1, keepdims=True)
        m3 = jnp.min(jnp.where(d <= m2, inf, d), axis=1, keepdims=True)
        r = jnp.where(d <= m3, 1.0 / (d + 1e-8), 0.0)
        return r / jnp.sum(r, axis=1, keepdims=True)

    TH = x1.shape[0] // 2
    y1p = jnp.dot(p1_ref[0], w1at_ref[...],
                  preferred_element_type=jnp.float32,
                  precision=jax.lax.Precision.HIGHEST)
    halves = []
    for h in range(2):
        wt = _weights(x1[h * TH:(h + 1) * TH])
        halves.append(jnp.dot(wt, p2w_scr[...],
                              preferred_element_type=jnp.float32,
                              precision=jax.lax.Precision.HIGHEST))
    y1 = y1p + jnp.concatenate(halves, axis=0) + b1_ref[...]
    y1_ref[0] = y1

    @pl.when((b == 0) & (n == 0))
    def _():
        s_ref[...] = jnp.zeros_like(s_ref)
        ss_ref[...] = jnp.zeros_like(ss_ref)

    s_ref[...] += jnp.sum(y1, axis=0, keepdims=True)
    ss_ref[...] += jnp.sum(y1 * y1, axis=0, keepdims=True)


def _bn_affine(s, ss, g, beta, minv):
    mean = s * minv
    var = ss * minv - mean * mean
    inv = jax.lax.rsqrt(var + 1e-5)
    scale = g * inv
    shift = beta - mean * scale
    return scale, shift


def _stage2_body(y1_ref, s_ref, ss_ref, g1_ref, bt1_ref, w2t_ref, b2_ref,
                 y2_ref, s2_ref, ss2_ref, *, minv):
    b = pl.program_id(0)
    n = pl.program_id(1)
    scale, shift = _bn_affine(s_ref[...], ss_ref[...], g1_ref[...],
                              bt1_ref[...], minv)
    h = jnp.maximum(y1_ref[0] * scale + shift, 0.0)
    y2 = jnp.dot(h, w2t_ref[...], preferred_element_type=jnp.float32, precision=jax.lax.Precision.HIGHEST)
    y2 = y2 + b2_ref[...]
    y2_ref[0] = y2

    @pl.when((b == 0) & (n == 0))
    def _():
        s2_ref[...] = jnp.zeros_like(s2_ref)
        ss2_ref[...] = jnp.zeros_like(ss2_ref)

    s2_ref[...] += jnp.sum(y2, axis=0, keepdims=True)
    ss2_ref[...] += jnp.sum(y2 * y2, axis=0, keepdims=True)


def _stage3_body(y2_ref, s2_ref, ss2_ref, g2_ref, bt2_ref, out_ref, *, minv):
    scale, shift = _bn_affine(s2_ref[...], ss2_ref[...], g2_ref[...],
                              bt2_ref[...], minv)
    out_ref[0] = jnp.maximum(y2_ref[0] * scale + shift, 0.0)


def kernel(xyz1, xyz2, points1, points2, W1, b1, g1, beta1, W2, b2, g2, beta2):
    B, N, _ = xyz1.shape
    S = xyz2.shape[1]
    C1 = points1.shape[2]
    C2 = points2.shape[2]
    F1 = W1.shape[0]
    F2 = W2.shape[0]
    TN = 1024
    NT = N // TN
    minv = 1.0 / (B * N)

    x2t = jnp.transpose(xyz2, (0, 2, 1))
    w1at = jnp.transpose(W1[:, :C1])
    w1bt = jnp.transpose(W1[:, C1:])
    w2t = jnp.transpose(W2)
    b1r = b1.reshape(1, F1)
    g1r = g1.reshape(1, F1)
    bt1r = beta1.reshape(1, F1)
    b2r = b2.reshape(1, F2)
    g2r = g2.reshape(1, F2)
    bt2r = beta2.reshape(1, F2)

    f32 = jnp.float32
    y1, s1, ss1 = pl.pallas_call(
        _stage1_body,
        grid=(B, NT),
        in_specs=[
            pl.BlockSpec((1, TN, 3), lambda b, n: (b, n, 0)),
            pl.BlockSpec((1, 3, S), lambda b, n: (b, 0, 0)),
            pl.BlockSpec((1, TN, C1), lambda b, n: (b, n, 0)),
            pl.BlockSpec((1, S, C2), lambda b, n: (b, 0, 0)),
            pl.BlockSpec((C1, F1), lambda b, n: (0, 0)),
            pl.BlockSpec((C2, F1), lambda b, n: (0, 0)),
            pl.BlockSpec((1, F1), lambda b, n: (0, 0)),
        ],
        out_specs=[
            pl.BlockSpec((1, TN, F1), lambda b, n: (b, n, 0)),
            pl.BlockSpec((1, F1), lambda b, n: (0, 0)),
            pl.BlockSpec((1, F1), lambda b, n: (0, 0)),
        ],
        out_shape=[
            jax.ShapeDtypeStruct((B, N, F1), f32),
            jax.ShapeDtypeStruct((1, F1), f32),
            jax.ShapeDtypeStruct((1, F1), f32),
        ],
        scratch_shapes=[pltpu.VMEM((S, F1), f32)],
    )(xyz1, x2t, points1, points2, w1at, w1bt, b1r)

    y2, s2, ss2 = pl.pallas_call(
        functools.partial(_stage2_body, minv=minv),
        grid=(B, NT),
        in_specs=[
            pl.BlockSpec((1, TN, F1), lambda b, n: (b, n, 0)),
            pl.BlockSpec((1, F1), lambda b, n: (0, 0)),
            pl.BlockSpec((1, F1), lambda b, n: (0, 0)),
            pl.BlockSpec((1, F1), lambda b, n: (0, 0)),
            pl.BlockSpec((1, F1), lambda b, n: (0, 0)),
            pl.BlockSpec((F1, F2), lambda b, n: (0, 0)),
            pl.BlockSpec((1, F2), lambda b, n: (0, 0)),
        ],
        out_specs=[
            pl.BlockSpec((1, TN, F2), lambda b, n: (b, n, 0)),
            pl.BlockSpec((1, F2), lambda b, n: (0, 0)),
            pl.BlockSpec((1, F2), lambda b, n: (0, 0)),
        ],
        out_shape=[
            jax.ShapeDtypeStruct((B, N, F2), f32),
            jax.ShapeDtypeStruct((1, F2), f32),
            jax.ShapeDtypeStruct((1, F2), f32),
        ],
    )(y1, s1, ss1, g1r, bt1r, w2t, b2r)

    out = pl.pallas_call(
        functools.partial(_stage3_body, minv=minv),
        grid=(B, NT),
        in_specs=[
            pl.BlockSpec((1, TN, F2), lambda b, n: (b, n, 0)),
            pl.BlockSpec((1, F2), lambda b, n: (0, 0)),
            pl.BlockSpec((1, F2), lambda b, n: (0, 0)),
            pl.BlockSpec((1, F2), lambda b, n: (0, 0)),
            pl.BlockSpec((1, F2), lambda b, n: (0, 0)),
        ],
        out_specs=pl.BlockSpec((1, TN, F2), lambda b, n: (b, n, 0)),
        out_shape=jax.ShapeDtypeStruct((B, N, F2), f32),
    )(y2, s2, ss2, g2r, bt2r)

    return out
